# baseline (speedup 1.0000x reference)
import jax
import jax.numpy as jnp
from jax.experimental import pallas as pl
from jax.experimental.pallas import tpu as pltpu

_TN = 1024
_SUB = 4
_CHUNK = 128


def _rvq_stage(r, embed_t, embed_t32, e2_row):
    tn, d = r.shape
    k = embed_t.shape[1]
    n_chunks = k // _CHUNK
    r2 = jnp.sum(r * r, axis=1, keepdims=True)
    scores = jax.lax.dot_general(
        r.astype(jnp.bfloat16), embed_t,
        (((1,), (0,)), ((), ())),
        preferred_element_type=jnp.float32)
    dist = (r2 - 2.0 * scores) + e2_row
    idx = jnp.argmin(dist, axis=1, keepdims=True).astype(jnp.int32)
    idx_l = jnp.swapaxes(idx, 0, 1)
    a_star = jnp.broadcast_to(idx_l & (_CHUNK - 1), (d, tn))
    c_star = jnp.broadcast_to(idx_l >> 7, (d, tn))
    quant_t = jnp.zeros((d, tn), dtype=jnp.float32)
    for c in range(n_chunks):
        src = embed_t32[:, c * _CHUNK:(c + 1) * _CHUNK]
        sel = jnp.take_along_axis(src, a_star, axis=1)
        quant_t = jnp.where(c_star == c, sel, quant_t)
    quant = jnp.swapaxes(quant_t, 0, 1)
    diff = quant - r
    return quant, idx, jnp.sum(diff * diff)


def _rvq_subtile(r, cbt_ref, cbt32_ref, e2_ref):
    nq = cbt_ref.shape[0]
    qout = jnp.zeros_like(r)
    idx_cols = []
    loss_vals = []
    for q in range(nq):
        quant, idx, lsum = _rvq_stage(
            r, cbt_ref[q], cbt32_ref[q], e2_ref[q:q + 1, :])
        loss_vals.append(lsum)
        idx_cols.append(idx)
        qout = qout + quant
        r = r - quant
    return qout, jnp.concatenate(idx_cols, axis=1), loss_vals


def _rvq_body(x_ref, cbt_ref, cbt32_ref, e2_ref, qout_ref, idx_ref, loss_ref):
    nq = cbt_ref.shape[0]
    tn = x_ref.shape[0]
    h = tn // _SUB
    loss_tot = None
    for s in range(_SUB):
        qs, is_, ls = _rvq_subtile(x_ref[s * h:(s + 1) * h],
                                   cbt_ref, cbt32_ref, e2_ref)
        qout_ref[s * h:(s + 1) * h, :] = qs
        idx_ref[s * h:(s + 1) * h, :] = is_
        loss_tot = ls if loss_tot is None else [a + b for a, b in
                                                zip(loss_tot, ls)]
    loss_ref[...] = jnp.stack(loss_tot).reshape(1, 1, nq)


def kernel(x, codebooks):
    B, N, D = x.shape
    NQ, K, _ = codebooks.shape
    T = B * N
    xf = x.reshape(T, D)
    e2 = jnp.sum(codebooks * codebooks, axis=-1)
    cbt32 = jnp.swapaxes(codebooks, 1, 2)
    cbt = cbt32.astype(jnp.bfloat16)
    n_tiles = T // _TN
    qout, idx, loss = pl.pallas_call(
        _rvq_body,
        grid=(n_tiles,),
        in_specs=[
            pl.BlockSpec((_TN, D), lambda i: (i, 0)),
            pl.BlockSpec((NQ, D, K), lambda i: (0, 0, 0)),
            pl.BlockSpec((NQ, D, K), lambda i: (0, 0, 0)),
            pl.BlockSpec((NQ, K), lambda i: (0, 0)),
        ],
        out_specs=[
            pl.BlockSpec((_TN, D), lambda i: (i, 0)),
            pl.BlockSpec((_TN, NQ), lambda i: (i, 0)),
            pl.BlockSpec((1, 1, NQ), lambda i: (i, 0, 0)),
        ],
        out_shape=[
            jax.ShapeDtypeStruct((T, D), jnp.float32),
            jax.ShapeDtypeStruct((T, NQ), jnp.int32),
            jax.ShapeDtypeStruct((n_tiles, 1, NQ), jnp.float32),
        ],
        compiler_params=pltpu.CompilerParams(
            dimension_semantics=("parallel",)),
    )(xf, cbt, cbt32, e2)
    quantized_out = qout.reshape(B, N, D)
    indices = idx.reshape(B, N, NQ)
    losses = jnp.sum(loss[:, 0, :], axis=0) / (B * N * D)
    return quantized_out, indices, losses


# speedup vs baseline: 1.3408x; 1.3408x over previous
import jax
import jax.numpy as jnp
from jax.experimental import pallas as pl
from jax.experimental.pallas import tpu as pltpu

_TN = 512
_SUB = 2
_CHUNK = 128


def _rvq_stage(r, embed_t, embed_t32, e2_row):
    tn, d = r.shape
    k = embed_t.shape[1]
    n_chunks = k // _CHUNK
    r2 = jnp.sum(r * r, axis=1, keepdims=True)
    scores2 = jax.lax.dot_general(
        (r + r).astype(jnp.bfloat16), embed_t,
        (((1,), (0,)), ((), ())),
        preferred_element_type=jnp.float32)
    dist = (r2 - scores2) + e2_row
    idx = jnp.argmin(dist, axis=1, keepdims=True).astype(jnp.int32)
    idx_l = jnp.swapaxes(idx, 0, 1)
    a_star = jnp.broadcast_to(idx_l & (_CHUNK - 1), (d, tn))
    c_star = jnp.broadcast_to(idx_l >> 7, (d, tn))
    quant_t = jnp.zeros((d, tn), dtype=jnp.float32)
    for c in range(n_chunks):
        src = embed_t32[:, c * _CHUNK:(c + 1) * _CHUNK]
        sel = jnp.take_along_axis(src, a_star, axis=1)
        quant_t = jnp.where(c_star == c, sel, quant_t)
    quant = jnp.swapaxes(quant_t, 0, 1)
    diff = quant - r
    return quant, idx, jnp.sum(diff * diff)


def _rvq_subtile(r, cbt_ref, cbt32_ref, e2_ref):
    nq = cbt_ref.shape[0]
    qout = jnp.zeros_like(r)
    idx_cols = []
    loss_vals = []
    for q in range(nq):
        quant, idx, lsum = _rvq_stage(
            r, cbt_ref[q], cbt32_ref[q], e2_ref[q:q + 1, :])
        loss_vals.append(lsum)
        idx_cols.append(idx)
        qout = qout + quant
        r = r - quant
    return qout, jnp.concatenate(idx_cols, axis=1), loss_vals


def _rvq_body(x_ref, cbt_ref, cbt32_ref, e2_ref, qout_ref, idx_ref, loss_ref):
    nq = cbt_ref.shape[0]
    tn = x_ref.shape[0]
    h = tn // _SUB
    loss_tot = None
    for s in range(_SUB):
        qs, is_, ls = _rvq_subtile(x_ref[s * h:(s + 1) * h],
                                   cbt_ref, cbt32_ref, e2_ref)
        qout_ref[s * h:(s + 1) * h, :] = qs
        idx_ref[s * h:(s + 1) * h, :] = is_
        loss_tot = ls if loss_tot is None else [a + b for a, b in
                                                zip(loss_tot, ls)]
    loss_ref[...] = jnp.stack(loss_tot).reshape(1, 1, nq)


def kernel(x, codebooks):
    B, N, D = x.shape
    NQ, K, _ = codebooks.shape
    T = B * N
    xf = x.reshape(T, D)
    e2 = jnp.sum(codebooks * codebooks, axis=-1)
    cbt32 = jnp.swapaxes(codebooks, 1, 2)
    cbt = cbt32.astype(jnp.bfloat16)
    n_tiles = T // _TN
    qout, idx, loss = pl.pallas_call(
        _rvq_body,
        grid=(n_tiles,),
        in_specs=[
            pl.BlockSpec((_TN, D), lambda i: (i, 0)),
            pl.BlockSpec((NQ, D, K), lambda i: (0, 0, 0)),
            pl.BlockSpec((NQ, D, K), lambda i: (0, 0, 0)),
            pl.BlockSpec((NQ, K), lambda i: (0, 0)),
        ],
        out_specs=[
            pl.BlockSpec((_TN, D), lambda i: (i, 0)),
            pl.BlockSpec((_TN, NQ), lambda i: (i, 0)),
            pl.BlockSpec((1, 1, NQ), lambda i: (i, 0, 0)),
        ],
        out_shape=[
            jax.ShapeDtypeStruct((T, D), jnp.float32),
            jax.ShapeDtypeStruct((T, NQ), jnp.int32),
            jax.ShapeDtypeStruct((n_tiles, 1, NQ), jnp.float32),
        ],
        compiler_params=pltpu.CompilerParams(
            dimension_semantics=("parallel",)),
    )(xf, cbt, cbt32, e2)
    quantized_out = qout.reshape(B, N, D)
    indices = idx.reshape(B, N, NQ)
    losses = jnp.sum(loss[:, 0, :], axis=0) / (B * N * D)
    return quantized_out, indices, losses
